# initial kernel scaffold (unmeasured)
import functools

import jax
import jax.numpy as jnp
from jax import lax
from jax.experimental import pallas as pl
from jax.experimental.pallas import tpu as pltpu

N_CHUNK = 8


def kernel(x, dy):
    m, d = x.shape
    _, f = dy.shape
    dh = d // 2
    fh = f // 2
    ch = fh // N_CHUNK

    my_x = lax.axis_index("x")
    my_y = lax.axis_index("y")

    dy_half = lax.dynamic_slice(dy, (0, my_y * fh), (m, fh))
    x_keep = lax.dynamic_slice(x, (0, my_x * dh), (m, dh))
    x_send = lax.dynamic_slice(x, (0, (1 - my_x) * dh), (m, dh))
    dn = (((0,), (0,)), ((), ()))
    q_keep = lax.dot_general(x_keep, dy_half, dn)
    q_send = lax.dot_general(x_send, dy_half, dn)

    def body(qk_ref, qs_ref, out_ref, s1, r1, s2, r2):
        mx = lax.axis_index("x")
        my = lax.axis_index("y")
        xn = (1 - mx, my)
        yn = (mx, 1 - my)

        barrier_sem = pltpu.get_barrier_semaphore()
        for nbr in (xn, yn):
            pl.semaphore_signal(
                barrier_sem, inc=1, device_id=nbr,
                device_id_type=pl.DeviceIdType.MESH,
            )
        pl.semaphore_wait(barrier_sem, 2)

        col0 = my * fh
        for c in range(N_CHUNK):
            cs = c * ch
            rdma1 = pltpu.make_async_remote_copy(
                src_ref=qs_ref.at[:, pl.ds(cs, ch)],
                dst_ref=out_ref.at[:, pl.ds(col0 + cs, ch)],
                send_sem=s1.at[c],
                recv_sem=r1.at[c],
                device_id=xn,
                device_id_type=pl.DeviceIdType.MESH,
            )
            rdma1.start()
            rdma1.wait()
            out_ref[:, pl.ds(col0 + cs, ch)] = (
                out_ref[:, pl.ds(col0 + cs, ch)] + qk_ref[:, pl.ds(cs, ch)]
            )
            rdma2 = pltpu.make_async_remote_copy(
                src_ref=out_ref.at[:, pl.ds(col0 + cs, ch)],
                dst_ref=out_ref.at[:, pl.ds(col0 + cs, ch)],
                send_sem=s2.at[c],
                recv_sem=r2.at[c],
                device_id=yn,
                device_id_type=pl.DeviceIdType.MESH,
            )
            rdma2.start()
            rdma2.wait()

        @functools.partial(
            pl.run_scoped, sem2=pltpu.SemaphoreType.REGULAR
        )
        def _(sem2):
            for nbr in (xn, yn):
                pl.semaphore_signal(
                    sem2, inc=1, device_id=nbr,
                    device_id_type=pl.DeviceIdType.MESH,
                )
            pl.semaphore_wait(sem2, 2)

    return pl.pallas_call(
        body,
        out_shape=jax.ShapeDtypeStruct((dh, f), jnp.float32),
        in_specs=[
            pl.BlockSpec(memory_space=pltpu.VMEM),
            pl.BlockSpec(memory_space=pltpu.ANY),
        ],
        out_specs=pl.BlockSpec(memory_space=pltpu.VMEM),
        scratch_shapes=[
            pltpu.SemaphoreType.DMA((N_CHUNK,)),
            pltpu.SemaphoreType.DMA((N_CHUNK,)),
            pltpu.SemaphoreType.DMA((N_CHUNK,)),
            pltpu.SemaphoreType.DMA((N_CHUNK,)),
        ],
        compiler_params=pltpu.CompilerParams(collective_id=0),
    )(q_keep, q_send)


# baseline (device time: 499469 ns/iter reference)
import functools

import jax
import jax.numpy as jnp
from jax import lax
from jax.experimental import pallas as pl
from jax.experimental.pallas import tpu as pltpu

N_CHUNK = 8


def kernel(x, dy):
    m, d = x.shape
    _, f = dy.shape
    dh = d // 2
    fh = f // 2
    ch = fh // N_CHUNK

    my_x = lax.axis_index("x")
    my_y = lax.axis_index("y")

    dy_half = lax.dynamic_slice(dy, (0, my_y * fh), (m, fh))
    x_keep = lax.dynamic_slice(x, (0, my_x * dh), (m, dh))
    x_send = lax.dynamic_slice(x, (0, (1 - my_x) * dh), (m, dh))
    dn = (((0,), (0,)), ((), ()))
    q_keep = lax.dot_general(x_keep, dy_half, dn)
    q_send = lax.dot_general(x_send, dy_half, dn)

    def body(qk_ref, qs_ref, out_ref, s1, r1, s2, r2):
        mx = lax.axis_index("x")
        my = lax.axis_index("y")
        xn = (1 - mx, my)
        yn = (mx, 1 - my)

        barrier_sem = pltpu.get_barrier_semaphore()
        for nbr in (xn, yn):
            pl.semaphore_signal(
                barrier_sem, inc=1, device_id=nbr,
                device_id_type=pl.DeviceIdType.MESH,
            )
        pl.semaphore_wait(barrier_sem, 2)

        col0 = my * fh
        for c in range(N_CHUNK):
            cs = c * ch
            rdma1 = pltpu.make_async_remote_copy(
                src_ref=qs_ref.at[:, pl.ds(cs, ch)],
                dst_ref=out_ref.at[:, pl.ds(col0 + cs, ch)],
                send_sem=s1.at[c],
                recv_sem=r1.at[c],
                device_id=xn,
                device_id_type=pl.DeviceIdType.MESH,
            )
            rdma1.start()
            rdma1.wait()
            out_ref[:, pl.ds(col0 + cs, ch)] = (
                out_ref[:, pl.ds(col0 + cs, ch)] + qk_ref[:, pl.ds(cs, ch)]
            )
            rdma2 = pltpu.make_async_remote_copy(
                src_ref=out_ref.at[:, pl.ds(col0 + cs, ch)],
                dst_ref=out_ref.at[:, pl.ds(col0 + cs, ch)],
                send_sem=s2.at[c],
                recv_sem=r2.at[c],
                device_id=yn,
                device_id_type=pl.DeviceIdType.MESH,
            )
            rdma2.start()
            rdma2.wait()

        @functools.partial(
            pl.run_scoped, sem2=pltpu.SemaphoreType.REGULAR
        )
        def _(sem2):
            for nbr in (xn, yn):
                pl.semaphore_signal(
                    sem2, inc=1, device_id=nbr,
                    device_id_type=pl.DeviceIdType.MESH,
                )
            pl.semaphore_wait(sem2, 2)

    return pl.pallas_call(
        body,
        out_shape=jax.ShapeDtypeStruct((dh, f), jnp.float32),
        in_specs=[
            pl.BlockSpec(memory_space=pltpu.VMEM),
            pl.BlockSpec(memory_space=pl.ANY),
        ],
        out_specs=pl.BlockSpec(memory_space=pltpu.VMEM),
        scratch_shapes=[
            pltpu.SemaphoreType.DMA((N_CHUNK,)),
            pltpu.SemaphoreType.DMA((N_CHUNK,)),
            pltpu.SemaphoreType.DMA((N_CHUNK,)),
            pltpu.SemaphoreType.DMA((N_CHUNK,)),
        ],
        compiler_params=pltpu.CompilerParams(collective_id=0),
    )(q_keep, q_send)


# device time: 315789 ns/iter; 1.5817x vs baseline; 1.5817x over previous
import functools

import jax
import jax.numpy as jnp
from jax import lax
from jax.experimental import pallas as pl
from jax.experimental.pallas import tpu as pltpu

N_CHUNK = 8


def kernel(x, dy):
    m, d = x.shape
    _, f = dy.shape
    dh = d // 2
    fh = f // 2
    ch = fh // N_CHUNK

    my_x = lax.axis_index("x")
    my_y = lax.axis_index("y")

    dy_half = lax.dynamic_slice(dy, (0, my_y * fh), (m, fh))
    x_keep = lax.dynamic_slice(x, (0, my_x * dh), (m, dh))
    x_send = lax.dynamic_slice(x, (0, (1 - my_x) * dh), (m, dh))
    dn = (((0,), (0,)), ((), ()))
    q_keep = lax.dot_general(x_keep, dy_half, dn)
    q_send = lax.dot_general(x_send, dy_half, dn)

    def body(qk_ref, qs_ref, out_ref, s1, r1, s2, r2):
        mx = lax.axis_index("x")
        my = lax.axis_index("y")
        xn = (1 - mx, my)
        yn = (mx, 1 - my)

        barrier_sem = pltpu.get_barrier_semaphore()
        for nbr in (xn, yn):
            pl.semaphore_signal(
                barrier_sem, inc=1, device_id=nbr,
                device_id_type=pl.DeviceIdType.MESH,
            )
        pl.semaphore_wait(barrier_sem, 2)

        col0 = my * fh

        rdma1s = []
        for c in range(N_CHUNK):
            cs = c * ch
            rdma1 = pltpu.make_async_remote_copy(
                src_ref=qs_ref.at[:, pl.ds(cs, ch)],
                dst_ref=out_ref.at[:, pl.ds(col0 + cs, ch)],
                send_sem=s1.at[c],
                recv_sem=r1.at[c],
                device_id=xn,
                device_id_type=pl.DeviceIdType.MESH,
            )
            rdma1.start()
            rdma1s.append(rdma1)

        rdma2s = []
        for c in range(N_CHUNK):
            cs = c * ch
            rdma1s[c].wait_recv()
            out_ref[:, pl.ds(col0 + cs, ch)] = (
                out_ref[:, pl.ds(col0 + cs, ch)] + qk_ref[:, pl.ds(cs, ch)]
            )
            rdma2 = pltpu.make_async_remote_copy(
                src_ref=out_ref.at[:, pl.ds(col0 + cs, ch)],
                dst_ref=out_ref.at[:, pl.ds(col0 + cs, ch)],
                send_sem=s2.at[c],
                recv_sem=r2.at[c],
                device_id=yn,
                device_id_type=pl.DeviceIdType.MESH,
            )
            rdma2.start()
            rdma2s.append(rdma2)

        for c in range(N_CHUNK):
            rdma2s[c].wait_recv()
        for c in range(N_CHUNK):
            rdma1s[c].wait_send()
            rdma2s[c].wait_send()

        @functools.partial(
            pl.run_scoped, sem2=pltpu.SemaphoreType.REGULAR
        )
        def _(sem2):
            for nbr in (xn, yn):
                pl.semaphore_signal(
                    sem2, inc=1, device_id=nbr,
                    device_id_type=pl.DeviceIdType.MESH,
                )
            pl.semaphore_wait(sem2, 2)

    return pl.pallas_call(
        body,
        out_shape=jax.ShapeDtypeStruct((dh, f), jnp.float32),
        in_specs=[
            pl.BlockSpec(memory_space=pltpu.VMEM),
            pl.BlockSpec(memory_space=pl.ANY),
        ],
        out_specs=pl.BlockSpec(memory_space=pltpu.VMEM),
        scratch_shapes=[
            pltpu.SemaphoreType.DMA((N_CHUNK,)),
            pltpu.SemaphoreType.DMA((N_CHUNK,)),
            pltpu.SemaphoreType.DMA((N_CHUNK,)),
            pltpu.SemaphoreType.DMA((N_CHUNK,)),
        ],
        compiler_params=pltpu.CompilerParams(collective_id=0),
    )(q_keep, q_send)


# device time: 241113 ns/iter; 2.0715x vs baseline; 1.3097x over previous
import functools

import jax
import jax.numpy as jnp
from jax import lax
from jax.experimental import pallas as pl
from jax.experimental.pallas import tpu as pltpu

N_CHUNK = 32
SLOTS = 4
LAG = 2


def kernel(x, dy):
    m, d = x.shape
    _, f = dy.shape
    dh = d // 2
    fh = f // 2
    ch = fh // N_CHUNK
    dn = (((0,), (0,)), ((), ()))

    def body(x_ref, dy_ref, out_ref, dy_buf, q_buf, dy_sems, s1, r1, s2, r2):
        mx = lax.axis_index("x")
        my = lax.axis_index("y")
        xn = (1 - mx, my)
        yn = (mx, 1 - my)
        col0 = my * fh
        keep0 = mx * dh
        send0 = (1 - mx) * dh

        def dy_dma(c):
            return pltpu.make_async_copy(
                dy_ref.at[:, pl.ds(col0 + c * ch, ch)],
                dy_buf.at[c % SLOTS],
                dy_sems.at[c % SLOTS],
            )

        dy_dmas = []
        for c in range(3):
            dma = dy_dma(c)
            dma.start()
            dy_dmas.append(dma)

        barrier_sem = pltpu.get_barrier_semaphore()
        for nbr in (xn, yn):
            pl.semaphore_signal(
                barrier_sem, inc=1, device_id=nbr,
                device_id_type=pl.DeviceIdType.MESH,
            )
        pl.semaphore_wait(barrier_sem, 2)

        rdma1s = []
        rdma2s = []

        def handle_arrival(j):
            rdma1s[j].wait_recv()
            out_ref[:, pl.ds(col0 + j * ch, ch)] = (
                out_ref[:, pl.ds(col0 + j * ch, ch)]
                + q_buf[j % SLOTS, pl.ds(keep0, dh), :]
            )
            rdma2 = pltpu.make_async_remote_copy(
                src_ref=out_ref.at[:, pl.ds(col0 + j * ch, ch)],
                dst_ref=out_ref.at[:, pl.ds(col0 + j * ch, ch)],
                send_sem=s2.at[j],
                recv_sem=r2.at[j],
                device_id=yn,
                device_id_type=pl.DeviceIdType.MESH,
            )
            rdma2.start()
            rdma2s.append(rdma2)

        for c in range(N_CHUNK):
            if c + 3 < N_CHUNK:
                dma = dy_dma(c + 3)
                dma.start()
                dy_dmas.append(dma)
            if c >= SLOTS:
                rdma1s[c - SLOTS].wait_send()
            dy_dmas[c].wait()
            q_buf[c % SLOTS] = lax.dot_general(
                x_ref[:, :], dy_buf[c % SLOTS], dn,
                preferred_element_type=jnp.float32,
            )
            rdma1 = pltpu.make_async_remote_copy(
                src_ref=q_buf.at[c % SLOTS, pl.ds(send0, dh), :],
                dst_ref=out_ref.at[:, pl.ds(col0 + c * ch, ch)],
                send_sem=s1.at[c],
                recv_sem=r1.at[c],
                device_id=xn,
                device_id_type=pl.DeviceIdType.MESH,
            )
            rdma1.start()
            rdma1s.append(rdma1)
            if c >= LAG:
                handle_arrival(c - LAG)

        for j in range(N_CHUNK - LAG, N_CHUNK):
            handle_arrival(j)

        for j in range(N_CHUNK):
            rdma2s[j].wait_recv()
        for j in range(N_CHUNK - SLOTS, N_CHUNK):
            rdma1s[j].wait_send()
        for j in range(N_CHUNK):
            rdma2s[j].wait_send()

        @functools.partial(
            pl.run_scoped, sem2=pltpu.SemaphoreType.REGULAR
        )
        def _(sem2):
            for nbr in (xn, yn):
                pl.semaphore_signal(
                    sem2, inc=1, device_id=nbr,
                    device_id_type=pl.DeviceIdType.MESH,
                )
            pl.semaphore_wait(sem2, 2)

    return pl.pallas_call(
        body,
        out_shape=jax.ShapeDtypeStruct((dh, f), jnp.float32),
        in_specs=[
            pl.BlockSpec(memory_space=pltpu.VMEM),
            pl.BlockSpec(memory_space=pl.ANY),
        ],
        out_specs=pl.BlockSpec(memory_space=pltpu.VMEM),
        scratch_shapes=[
            pltpu.VMEM((SLOTS, m, ch), jnp.float32),
            pltpu.VMEM((SLOTS, d, ch), jnp.float32),
            pltpu.SemaphoreType.DMA((SLOTS,)),
            pltpu.SemaphoreType.DMA((N_CHUNK,)),
            pltpu.SemaphoreType.DMA((N_CHUNK,)),
            pltpu.SemaphoreType.DMA((N_CHUNK,)),
            pltpu.SemaphoreType.DMA((N_CHUNK,)),
        ],
        compiler_params=pltpu.CompilerParams(
            collective_id=0, vmem_limit_bytes=64 * 1024 * 1024
        ),
    )(x, dy)
